# baseline (speedup 1.0000x reference)
import functools

import jax
import jax.numpy as jnp
import numpy as np
from jax import lax
from jax.experimental import pallas as pl
from jax.experimental.pallas import tpu as pltpu
from jax.experimental.pallas import tpu_sc as plsc

_MASK_RATIO = 0.75
_INT_MIN = np.int32(-2147483648)
_L = 16


def _row_select_body(imp_hbm, out_hbm, key_v, out_v, idx_v, hist_v,
                     red_v, part_v, shared, *, hw, len_keep):
    half_n = hw // 2
    nv = half_n // _L
    c = lax.axis_index("c")
    s = lax.axis_index("s")
    row = c * 8 + (s >> 1)
    half = s & 1
    base = half * half_n

    pltpu.sync_copy(imp_hbm.at[row, pl.ds(base, half_n)], key_v)

    lane = lax.iota(jnp.int32, _L)
    lane_base = lane * np.int32(256)
    ones = jnp.full((_L,), 1, jnp.int32)
    zeros16 = jnp.zeros((_L,), jnp.int32)

    def zero_hist(j, _):
        hist_v[pl.ds(j * _L, _L)] = zeros16
        return 0

    def to_key(b):
        m = b >> 31
        return (b ^ (m | _INT_MIN)) - m

    def exchange_and_select(p, prefix, rem):
        def reduce_copies(j, _):
            acc = hist_v[pl.ds(j * _L, _L)]
            for l in range(1, _L):
                acc = acc + hist_v[pl.ds(l * 256 + j * _L, _L)]
            red_v[pl.ds(j * _L, _L)] = acc
            return 0

        lax.fori_loop(0, 16, reduce_copies, 0, unroll=4)
        pltpu.sync_copy(red_v, shared.at[p, s])
        plsc.subcore_barrier()
        pltpu.sync_copy(shared.at[p, s ^ 1], part_v)

        def select(j, sc):
            nlt, below, off = sc
            acc = red_v[pl.ds(j * _L, _L)] + part_v[pl.ds(j * _L, _L)]
            cum = off + plsc.cumsum(acc)
            m = cum < rem
            nlt = nlt + jnp.sum(m.astype(jnp.int32))
            below = below + jnp.sum(jnp.where(m, acc, 0))
            off = off + jnp.sum(acc)
            return nlt, below, off

        bkt, below, _ = lax.fori_loop(0, 16, select,
                                      (jnp.int32(0), jnp.int32(0),
                                       jnp.int32(0)))
        return (prefix << 8) | bkt, rem - below, bkt

    lax.fori_loop(0, 256, zero_hist, 0, unroll=8)

    def rot_base(i):
        return ((lane + i) & np.int32(15)) * np.int32(256)

    def scan0(i, rb):
        ku = to_key(key_v[pl.ds(i * _L, _L)])
        key_v[pl.ds(i * _L, _L)] = ku
        bucket = lax.shift_right_logical(ku, 24)
        plsc.addupdate_scatter(hist_v, [rb + bucket], ones)
        return (rb + np.int32(256)) & np.int32(4095)

    lax.fori_loop(0, nv, scan0, lane_base)
    prefix, rem, bkt = exchange_and_select(0, jnp.int32(0),
                                           jnp.int32(len_keep))

    lax.fori_loop(0, 256, zero_hist, 0, unroll=8)

    def scan1(i, carry):
        wpos, rb = carry
        ku = key_v[pl.ds(i * _L, _L)]
        bucket_hi = lax.shift_right_logical(ku, 24)
        out_v[pl.ds(i * _L, _L)] = jnp.where(bucket_hi < prefix, 0.0, 1.0)
        active = bucket_hi == prefix
        bucket = lax.shift_right_logical(ku, 16) & np.int32(0xFF)
        plsc.addupdate_scatter(hist_v, [rb + bucket], ones,
                               mask=active)
        plsc.store_compressed(idx_v.at[pl.ds(wpos, _L)], i * _L + lane,
                              mask=active)
        return (wpos + plsc.all_reduce_population_count(active)[0],
                (rb + np.int32(256)) & np.int32(4095))

    nc, _ = lax.fori_loop(0, nv, scan1, (jnp.int32(0), lane_base))
    prefix, rem, bkt = exchange_and_select(1, prefix, rem)
    nvc = (nc + np.int32(_L - 1)) >> 4

    def gather_keys(j):
        valid = (j * _L + lane) < nc
        idx = jnp.where(valid, idx_v[pl.ds(j * _L, _L)], 0)
        ku = plsc.load_gather(key_v, [idx])
        return idx, ku, valid

    def compact_pass(p, prefix, rem):
        shift = 24 - 8 * p
        lax.fori_loop(0, 256, zero_hist, 0, unroll=8)

        def scanc(j, _):
            _, ku, valid = gather_keys(j)
            active = (lax.shift_right_logical(ku, shift + 8) == prefix) \
                & valid
            bucket = lax.shift_right_logical(ku, shift) & np.int32(0xFF)
            plsc.addupdate_scatter(hist_v, [rot_base(j) + bucket], ones,
                                   mask=active)
            return 0

        lax.fori_loop(0, nvc, scanc, 0)
        return exchange_and_select(p, prefix, rem)

    prefix, rem, bkt = compact_pass(2, prefix, rem)
    prefix, rem, bkt = compact_pass(3, prefix, rem)

    t_ku = prefix
    t_ks = prefix ^ _INT_MIN

    pvec = part_v[pl.ds((bkt >> 4) * _L, _L)]
    peq = jnp.sum(jnp.where(lane == (bkt & 15), pvec, 0))
    running0 = jnp.where(half == 1, peq, jnp.int32(0))

    def fix(j, carry):
        idx, ku, valid = gather_keys(j)
        eq = ku == t_ku
        eqi = (eq & valid).astype(jnp.int32)
        cume = plsc.cumsum(eqi) + carry
        zero = ((ku ^ _INT_MIN) < t_ks) | (eq & (cume <= rem))
        plsc.store_scatter(out_v, [idx],
                           jnp.where(zero, 0.0, 1.0), mask=valid)
        return carry + jnp.sum(eqi)

    lax.fori_loop(0, nvc, fix, running0)
    pltpu.sync_copy(out_v, out_hbm.at[row, pl.ds(base, half_n)])


def kernel(image, importance):
    n, c, h, w = image.shape
    hw = h * w
    len_keep = int(hw * (1 - _MASK_RATIO))
    imp = lax.bitcast_convert_type(importance.reshape(n, hw), jnp.int32)

    body = functools.partial(_row_select_body, hw=hw, len_keep=len_keep)
    mask = pl.kernel(
        body,
        out_type=jax.ShapeDtypeStruct((n, hw), jnp.float32),
        mesh=plsc.VectorSubcoreMesh(core_axis_name="c", subcore_axis_name="s"),
        compiler_params=pltpu.CompilerParams(needs_layout_passes=False),
        scratch_types=[
            pltpu.VMEM((hw // 2,), jnp.int32),
            pltpu.VMEM((hw // 2,), jnp.float32),
            pltpu.VMEM((hw // 2,), jnp.int32),
            pltpu.VMEM((_L * 256,), jnp.int32),
            pltpu.VMEM((256,), jnp.int32),
            pltpu.VMEM((256,), jnp.int32),
            pltpu.VMEM_SHARED((4, 16, 256), jnp.int32),
        ],
    )(imp)
    return mask.reshape(n, 1, h, w)


# speedup vs baseline: 1.0241x; 1.0241x over previous
import functools

import jax
import jax.numpy as jnp
import numpy as np
from jax import lax
from jax.experimental import pallas as pl
from jax.experimental.pallas import tpu as pltpu
from jax.experimental.pallas import tpu_sc as plsc

_MASK_RATIO = 0.75
_INT_MIN = np.int32(-2147483648)
_L = 16


def _row_select_body(imp_hbm, out_hbm, key_v, out_v, idx_v, hist_v,
                     red_v, part_v, shared, *, hw, len_keep):
    half_n = hw // 2
    nv = half_n // _L
    c = lax.axis_index("c")
    s = lax.axis_index("s")
    row = c * 8 + (s >> 1)
    half = s & 1
    base = half * half_n

    pltpu.sync_copy(imp_hbm.at[row, pl.ds(base, half_n)], key_v)

    lane = lax.iota(jnp.int32, _L)
    lane_base = lane * np.int32(256)
    ones = jnp.full((_L,), 1, jnp.int32)
    zeros16 = jnp.zeros((_L,), jnp.int32)

    def zero_hist(j, _):
        hist_v[pl.ds(j * _L, _L)] = zeros16
        return 0

    def to_key(b):
        m = b >> 31
        return (b ^ (m | _INT_MIN)) - m

    def exchange_and_select(p, prefix, rem):
        def reduce_copies(j, _):
            acc = hist_v[pl.ds(j * _L, _L)]
            for l in range(1, _L):
                acc = acc + hist_v[pl.ds(l * 256 + j * _L, _L)]
            red_v[pl.ds(j * _L, _L)] = acc
            return 0

        lax.fori_loop(0, 16, reduce_copies, 0, unroll=4)
        pltpu.sync_copy(red_v, shared.at[p, s])
        plsc.subcore_barrier()
        pltpu.sync_copy(shared.at[p, s ^ 1], part_v)

        def select(j, sc):
            nlt, below, off = sc
            acc = red_v[pl.ds(j * _L, _L)] + part_v[pl.ds(j * _L, _L)]
            cum = off + plsc.cumsum(acc)
            m = cum < rem
            nlt = nlt + jnp.sum(m.astype(jnp.int32))
            below = below + jnp.sum(jnp.where(m, acc, 0))
            off = off + jnp.sum(acc)
            return nlt, below, off

        bkt, below, _ = lax.fori_loop(0, 16, select,
                                      (jnp.int32(0), jnp.int32(0),
                                       jnp.int32(0)))
        return (prefix << 8) | bkt, rem - below, bkt

    lax.fori_loop(0, 256, zero_hist, 0, unroll=8)

    def rot_base(i):
        return ((lane + i) & np.int32(15)) * np.int32(256)

    def scan0(i, _):
        ku = to_key(key_v[pl.ds(i * _L, _L)])
        key_v[pl.ds(i * _L, _L)] = ku
        bucket = lax.shift_right_logical(ku, 24)
        plsc.addupdate_scatter(hist_v, [rot_base(i) + bucket], ones)
        return 0

    lax.fori_loop(0, nv, scan0, 0, unroll=2)
    prefix, rem, bkt = exchange_and_select(0, jnp.int32(0),
                                           jnp.int32(len_keep))

    lax.fori_loop(0, 256, zero_hist, 0, unroll=8)

    def scan1(i, wpos):
        ku = key_v[pl.ds(i * _L, _L)]
        bucket_hi = lax.shift_right_logical(ku, 24)
        out_v[pl.ds(i * _L, _L)] = jnp.where(bucket_hi < prefix, 0.0, 1.0)
        active = bucket_hi == prefix
        bucket = lax.shift_right_logical(ku, 16) & np.int32(0xFF)
        plsc.addupdate_scatter(hist_v, [rot_base(i) + bucket], ones,
                               mask=active)
        plsc.store_compressed(idx_v.at[pl.ds(wpos, _L)], i * _L + lane,
                              mask=active)
        return wpos + plsc.all_reduce_population_count(active)[0]

    nc = lax.fori_loop(0, nv, scan1, jnp.int32(0), unroll=2)
    prefix, rem, bkt = exchange_and_select(1, prefix, rem)
    nvc = (nc + np.int32(_L - 1)) >> 4

    def gather_keys(j):
        valid = (j * _L + lane) < nc
        idx = jnp.where(valid, idx_v[pl.ds(j * _L, _L)], 0)
        ku = plsc.load_gather(key_v, [idx])
        return idx, ku, valid

    def compact_pass(p, prefix, rem):
        shift = 24 - 8 * p
        lax.fori_loop(0, 256, zero_hist, 0, unroll=8)

        def scanc(j, _):
            _, ku, valid = gather_keys(j)
            active = (lax.shift_right_logical(ku, shift + 8) == prefix) \
                & valid
            bucket = lax.shift_right_logical(ku, shift) & np.int32(0xFF)
            plsc.addupdate_scatter(hist_v, [rot_base(j) + bucket], ones,
                                   mask=active)
            return 0

        lax.fori_loop(0, nvc, scanc, 0)
        return exchange_and_select(p, prefix, rem)

    prefix, rem, bkt = compact_pass(2, prefix, rem)
    prefix, rem, bkt = compact_pass(3, prefix, rem)

    t_ku = prefix
    t_ks = prefix ^ _INT_MIN

    pvec = part_v[pl.ds((bkt >> 4) * _L, _L)]
    peq = jnp.sum(jnp.where(lane == (bkt & 15), pvec, 0))
    running0 = jnp.where(half == 1, peq, jnp.int32(0))

    def fix(j, carry):
        idx, ku, valid = gather_keys(j)
        eq = ku == t_ku
        eqi = (eq & valid).astype(jnp.int32)
        cume = plsc.cumsum(eqi) + carry
        zero = ((ku ^ _INT_MIN) < t_ks) | (eq & (cume <= rem))
        plsc.store_scatter(out_v, [idx],
                           jnp.where(zero, 0.0, 1.0), mask=valid)
        return carry + jnp.sum(eqi)

    lax.fori_loop(0, nvc, fix, running0)
    pltpu.sync_copy(out_v, out_hbm.at[row, pl.ds(base, half_n)])


def kernel(image, importance):
    n, c, h, w = image.shape
    hw = h * w
    len_keep = int(hw * (1 - _MASK_RATIO))
    imp = lax.bitcast_convert_type(importance.reshape(n, hw), jnp.int32)

    body = functools.partial(_row_select_body, hw=hw, len_keep=len_keep)
    mask = pl.kernel(
        body,
        out_type=jax.ShapeDtypeStruct((n, hw), jnp.float32),
        mesh=plsc.VectorSubcoreMesh(core_axis_name="c", subcore_axis_name="s"),
        compiler_params=pltpu.CompilerParams(needs_layout_passes=False),
        scratch_types=[
            pltpu.VMEM((hw // 2,), jnp.int32),
            pltpu.VMEM((hw // 2,), jnp.float32),
            pltpu.VMEM((hw // 2,), jnp.int32),
            pltpu.VMEM((_L * 256,), jnp.int32),
            pltpu.VMEM((256,), jnp.int32),
            pltpu.VMEM((256,), jnp.int32),
            pltpu.VMEM_SHARED((4, 16, 256), jnp.int32),
        ],
    )(imp)
    return mask.reshape(n, 1, h, w)


# speedup vs baseline: 1.0256x; 1.0014x over previous
import functools

import jax
import jax.numpy as jnp
import numpy as np
from jax import lax
from jax.experimental import pallas as pl
from jax.experimental.pallas import tpu as pltpu
from jax.experimental.pallas import tpu_sc as plsc

_MASK_RATIO = 0.75
_INT_MIN = np.int32(-2147483648)
_L = 16


def _row_select_body(imp_hbm, out_hbm, key_v, out_v, idx_v, hist_v,
                     red_v, part_v, shared, *, hw, len_keep):
    half_n = hw // 2
    nv = half_n // _L
    c = lax.axis_index("c")
    s = lax.axis_index("s")
    row = c * 8 + (s >> 1)
    half = s & 1
    base = half * half_n

    pltpu.sync_copy(imp_hbm.at[row, pl.ds(base, half_n)], key_v)

    lane = lax.iota(jnp.int32, _L)
    lane_base = lane * np.int32(256)
    ones = jnp.full((_L,), 1, jnp.int32)
    zeros16 = jnp.zeros((_L,), jnp.int32)

    def zero_hist(j, _):
        hist_v[pl.ds(j * _L, _L)] = zeros16
        return 0

    def to_key(b):
        m = b >> 31
        return (b ^ (m | _INT_MIN)) - m

    def exchange_and_select(p, prefix, rem):
        def reduce_copies(j, _):
            acc = hist_v[pl.ds(j * _L, _L)]
            for l in range(1, _L):
                acc = acc + hist_v[pl.ds(l * 256 + j * _L, _L)]
            red_v[pl.ds(j * _L, _L)] = acc
            return 0

        lax.fori_loop(0, 16, reduce_copies, 0, unroll=4)
        pltpu.sync_copy(red_v, shared.at[p, s])
        plsc.subcore_barrier()
        pltpu.sync_copy(shared.at[p, s ^ 1], part_v)

        def select(j, sc):
            nlt, below, off = sc
            acc = red_v[pl.ds(j * _L, _L)] + part_v[pl.ds(j * _L, _L)]
            cum = off + plsc.cumsum(acc)
            m = cum < rem
            nlt = nlt + jnp.sum(m.astype(jnp.int32))
            below = below + jnp.sum(jnp.where(m, acc, 0))
            off = off + jnp.sum(acc)
            return nlt, below, off

        bkt, below, _ = lax.fori_loop(0, 16, select,
                                      (jnp.int32(0), jnp.int32(0),
                                       jnp.int32(0)))
        return (prefix << 8) | bkt, rem - below, bkt

    lax.fori_loop(0, 256, zero_hist, 0, unroll=8)

    def rot_base(i):
        return ((lane + i) & np.int32(15)) * np.int32(256)

    def scan0(i, _):
        ku = to_key(key_v[pl.ds(i * _L, _L)])
        key_v[pl.ds(i * _L, _L)] = ku
        bucket = lax.shift_right_logical(ku, 24)
        plsc.addupdate_scatter(hist_v, [rot_base(i) + bucket], ones)
        return 0

    lax.fori_loop(0, nv, scan0, 0, unroll=4)
    prefix, rem, bkt = exchange_and_select(0, jnp.int32(0),
                                           jnp.int32(len_keep))

    lax.fori_loop(0, 256, zero_hist, 0, unroll=8)

    def scan1(i, wpos):
        ku = key_v[pl.ds(i * _L, _L)]
        bucket_hi = lax.shift_right_logical(ku, 24)
        out_v[pl.ds(i * _L, _L)] = jnp.where(bucket_hi < prefix, 0.0, 1.0)
        active = bucket_hi == prefix
        bucket = lax.shift_right_logical(ku, 16) & np.int32(0xFF)
        plsc.addupdate_scatter(hist_v, [rot_base(i) + bucket], ones,
                               mask=active)
        plsc.store_compressed(idx_v.at[pl.ds(wpos, _L)], i * _L + lane,
                              mask=active)
        return wpos + plsc.all_reduce_population_count(active)[0]

    nc = lax.fori_loop(0, nv, scan1, jnp.int32(0), unroll=4)
    prefix, rem, bkt = exchange_and_select(1, prefix, rem)
    nvc = (nc + np.int32(_L - 1)) >> 4

    def gather_keys(j):
        valid = (j * _L + lane) < nc
        idx = jnp.where(valid, idx_v[pl.ds(j * _L, _L)], 0)
        ku = plsc.load_gather(key_v, [idx])
        return idx, ku, valid

    def compact_pass(p, prefix, rem):
        shift = 24 - 8 * p
        lax.fori_loop(0, 256, zero_hist, 0, unroll=8)

        def scanc(j, _):
            _, ku, valid = gather_keys(j)
            active = (lax.shift_right_logical(ku, shift + 8) == prefix) \
                & valid
            bucket = lax.shift_right_logical(ku, shift) & np.int32(0xFF)
            plsc.addupdate_scatter(hist_v, [rot_base(j) + bucket], ones,
                                   mask=active)
            return 0

        lax.fori_loop(0, nvc, scanc, 0)
        return exchange_and_select(p, prefix, rem)

    prefix, rem, bkt = compact_pass(2, prefix, rem)
    prefix, rem, bkt = compact_pass(3, prefix, rem)

    t_ku = prefix
    t_ks = prefix ^ _INT_MIN

    pvec = part_v[pl.ds((bkt >> 4) * _L, _L)]
    peq = jnp.sum(jnp.where(lane == (bkt & 15), pvec, 0))
    running0 = jnp.where(half == 1, peq, jnp.int32(0))

    def fix(j, carry):
        idx, ku, valid = gather_keys(j)
        eq = ku == t_ku
        eqi = (eq & valid).astype(jnp.int32)
        cume = plsc.cumsum(eqi) + carry
        zero = ((ku ^ _INT_MIN) < t_ks) | (eq & (cume <= rem))
        plsc.store_scatter(out_v, [idx],
                           jnp.where(zero, 0.0, 1.0), mask=valid)
        return carry + jnp.sum(eqi)

    lax.fori_loop(0, nvc, fix, running0)
    pltpu.sync_copy(out_v, out_hbm.at[row, pl.ds(base, half_n)])


def kernel(image, importance):
    n, c, h, w = image.shape
    hw = h * w
    len_keep = int(hw * (1 - _MASK_RATIO))
    imp = lax.bitcast_convert_type(importance.reshape(n, hw), jnp.int32)

    body = functools.partial(_row_select_body, hw=hw, len_keep=len_keep)
    mask = pl.kernel(
        body,
        out_type=jax.ShapeDtypeStruct((n, hw), jnp.float32),
        mesh=plsc.VectorSubcoreMesh(core_axis_name="c", subcore_axis_name="s"),
        compiler_params=pltpu.CompilerParams(needs_layout_passes=False),
        scratch_types=[
            pltpu.VMEM((hw // 2,), jnp.int32),
            pltpu.VMEM((hw // 2,), jnp.float32),
            pltpu.VMEM((hw // 2,), jnp.int32),
            pltpu.VMEM((_L * 256,), jnp.int32),
            pltpu.VMEM((256,), jnp.int32),
            pltpu.VMEM((256,), jnp.int32),
            pltpu.VMEM_SHARED((4, 16, 256), jnp.int32),
        ],
    )(imp)
    return mask.reshape(n, 1, h, w)
